# baseline (speedup 1.0000x reference)
import functools

import jax
import jax.numpy as jnp
from jax import lax
from jax.experimental import pallas as pl
from jax.experimental.pallas import tpu as pltpu
from jax.experimental.pallas import tpu_sc as plsc

N_SENS = 4760
N_PAD = 5120
TILE = 512
NT = N_PAD // TILE
KNN = 16
HID = 64
PRJ = 128
NH = 4
HD = 32
NHALF = N_PAD // 2


def _prep(q_ref, wql_ref, bql_ref, wqg_ref, bqg_ref, ws_ref, bnbr_ref,
          ql_out, qg_out, q2_out):
    q = q_ref[...]
    ql_out[...] = q @ wql_ref[...] + bql_ref[...]
    qg_out[...] = q @ wqg_ref[...] + bqg_ref[...]
    q2_out[...] = q @ ws_ref[...] + bnbr_ref[...]


def _sc_gather(x0_hbm, x1_hbm, em_hbm, idx_hbm,
               gx0_hbm, gx1_hbm, gm_hbm,
               idx_v, s0_v, s1_v, s2_v, out_v):
    c = lax.axis_index("c")
    s = lax.axis_index("s")
    wid = s * 2 + c
    b = wid // 2
    n0 = (wid % 2) * NHALF
    pltpu.sync_copy(idx_hbm.at[:, pl.ds(n0, NHALF)], idx_v)
    pltpu.sync_copy(x0_hbm.at[b], s0_v)
    pltpu.sync_copy(x1_hbm.at[b], s1_v)
    pltpu.sync_copy(em_hbm.at[b], s2_v)

    def make_body(src_v, k):
        def body(j, carry):
            iv = idx_v[k, pl.ds(j * 16, 16)]
            out_v[k, pl.ds(j * 16, 16)] = plsc.load_gather(src_v, [iv])
            return carry
        return body

    for src_v, dst_hbm in ((s0_v, gx0_hbm), (s1_v, gx1_hbm), (s2_v, gm_hbm)):
        for k in range(KNN):
            lax.fori_loop(0, NHALF // 16, make_body(src_v, k), 0)
        pltpu.sync_copy(out_v, dst_hbm.at[b, :, pl.ds(n0, NHALF)])


def _main(gx0_ref, gx1_ref, gm_ref, q2g_ref, qlt_ref, qg_ref, lat_ref, msk_ref,
          wxt_ref, wlat_ref, blat_ref, femb_ref, wlf_ref, blf_ref,
          wk_ref, bk_ref, wv_ref, bv_ref, wgo_ref, bgo_ref,
          lng_ref, lnb_ref, wm1_ref, bm1_ref, wm2_ref, bm2_ref,
          out_ref):
    qlt = qlt_ref[...]
    w0c = wxt_ref[:, 0:1]
    w1c = wxt_ref[:, 1:2]
    a0 = jnp.sum(qlt * w0c, axis=0, keepdims=True)
    a1 = jnp.sum(qlt * w1c, axis=0, keepdims=True)
    rows = []
    for k in range(KNN):
        rows.append(jnp.sum(q2g_ref[k] * qlt, axis=0, keepdims=True))
    dq = jnp.concatenate(rows, axis=0)
    gx0 = gx0_ref[0]
    gx1 = gx1_ref[0]
    lg = (gx0 * a0 + gx1 * a1 + dq) * (HID ** -0.5)
    lg = jnp.where(gm_ref[0] > 0, -10000.0, lg)
    mx = jnp.max(lg, axis=0, keepdims=True)
    ex = jnp.exp(lg - mx)
    aw = ex / jnp.sum(ex, axis=0, keepdims=True)
    s0 = jnp.sum(aw * gx0, axis=0, keepdims=True)
    s1 = jnp.sum(aw * gx1, axis=0, keepdims=True)
    localt = w0c * s0 + w1c * s1
    for k in range(KNN):
        localt = localt + aw[k:k + 1, :] * q2g_ref[k]
    local = localt.T

    lat = lat_ref[0]
    lfb = femb_ref[...] @ wlf_ref[...] + blf_ref[...]
    kv = lat @ wlat_ref[...] + blat_ref[...] + lfb
    kg = kv @ wk_ref[...] + bk_ref[...]
    vg = kv @ wv_ref[...] + bv_ref[...]
    qg = qg_ref[...]
    heads = []
    for h in range(NH):
        qh = qg[:, HD * h:HD * (h + 1)]
        kh = kg[:, HD * h:HD * (h + 1)]
        vh = vg[:, HD * h:HD * (h + 1)]
        s = lax.dot_general(qh, kh, (((1,), (1,)), ((), ()))) * (HD ** -0.5)
        s = s - jnp.max(s, axis=1, keepdims=True)
        es = jnp.exp(s)
        hw = es / jnp.sum(es, axis=1, keepdims=True)
        heads.append(hw @ vh)
    gf = jnp.concatenate(heads, axis=1)
    gf = gf @ wgo_ref[...] + bgo_ref[...]

    comb = jnp.concatenate([local, gf], axis=1)
    mu = jnp.mean(comb, axis=1, keepdims=True)
    var = jnp.mean((comb - mu) * (comb - mu), axis=1, keepdims=True)
    xn = (comb - mu) * jax.lax.rsqrt(var + 1e-5) * lng_ref[...] + lnb_ref[...]
    hm = xn @ wm1_ref[...] + bm1_ref[...]
    hm = 0.5 * hm * (1.0 + jax.lax.erf(hm * (2.0 ** -0.5)))
    pr = hm @ wm2_ref[...] + bm2_ref[...]
    out_ref[0] = pr * msk_ref[0]


def _full(shape):
    nd = len(shape)
    return pl.BlockSpec(shape, lambda t, b, _n=nd: (0,) * _n)


def kernel(x_flat, latent_seq, mask, encoder_mask, pos_embed, knn_indices,
           face_ids, token_face_ids, face_emb, W_nbr, b_nbr, W_ql, b_ql,
           W_lat, b_lat, W_lf, b_lf, W_qg, b_qg, W_k, b_k, W_v, b_v,
           W_go, b_go, ln_g, ln_b, W_m1, b_m1, W_m2, b_m2):
    B = x_flat.shape[0]
    pad = N_PAD - N_SENS

    query = jnp.concatenate([pos_embed, face_emb[face_ids]], axis=-1)
    query = jnp.pad(query, ((0, pad), (0, 0)))

    ql, qg, q2 = pl.pallas_call(
        _prep,
        grid=(NT,),
        in_specs=[
            pl.BlockSpec((TILE, 128), lambda t: (t, 0)),
            pl.BlockSpec((128, HID), lambda t: (0, 0)),
            pl.BlockSpec((1, HID), lambda t: (0, 0)),
            pl.BlockSpec((128, PRJ), lambda t: (0, 0)),
            pl.BlockSpec((1, PRJ), lambda t: (0, 0)),
            pl.BlockSpec((128, HID), lambda t: (0, 0)),
            pl.BlockSpec((1, HID), lambda t: (0, 0)),
        ],
        out_specs=[
            pl.BlockSpec((TILE, HID), lambda t: (t, 0)),
            pl.BlockSpec((TILE, PRJ), lambda t: (t, 0)),
            pl.BlockSpec((TILE, HID), lambda t: (t, 0)),
        ],
        out_shape=[
            jax.ShapeDtypeStruct((N_PAD, HID), jnp.float32),
            jax.ShapeDtypeStruct((N_PAD, PRJ), jnp.float32),
            jax.ShapeDtypeStruct((N_PAD, HID), jnp.float32),
        ],
    )(query, W_ql, b_ql.reshape(1, HID), W_qg, b_qg.reshape(1, PRJ),
      W_nbr[2:], b_nbr.reshape(1, HID))

    idxt = jnp.pad(knn_indices, ((0, pad), (0, 0))).T
    x0 = jnp.pad(x_flat[..., 0], ((0, 0), (0, pad)))
    x1 = jnp.pad(x_flat[..., 1], ((0, 0), (0, pad)))
    em = jnp.pad(encoder_mask, ((0, 0), (0, pad)))

    sc_gather = functools.partial(
        pl.kernel,
        out_type=[
            jax.ShapeDtypeStruct((B, KNN, N_PAD), jnp.float32),
            jax.ShapeDtypeStruct((B, KNN, N_PAD), jnp.float32),
            jax.ShapeDtypeStruct((B, KNN, N_PAD), jnp.float32),
        ],
        mesh=plsc.VectorSubcoreMesh(core_axis_name="c", subcore_axis_name="s"),
        compiler_params=pltpu.CompilerParams(needs_layout_passes=False),
        scratch_types=[
            pltpu.VMEM((KNN, NHALF), jnp.int32),
            pltpu.VMEM((N_PAD,), jnp.float32),
            pltpu.VMEM((N_PAD,), jnp.float32),
            pltpu.VMEM((N_PAD,), jnp.float32),
            pltpu.VMEM((KNN, NHALF), jnp.float32),
        ],
    )(_sc_gather)
    gx0, gx1, gm = sc_gather(x0, x1, em, idxt)

    q2g = jnp.transpose(jnp.take(q2, idxt, axis=0), (0, 2, 1))
    qlt = ql.T

    mcol = jnp.pad(mask, ((0, 0), (0, pad)))[..., None]
    mcol = (mcol > 0).astype(jnp.float32)

    out = pl.pallas_call(
        _main,
        grid=(NT, B),
        in_specs=[
            pl.BlockSpec((1, KNN, TILE), lambda t, b: (b, 0, t)),
            pl.BlockSpec((1, KNN, TILE), lambda t, b: (b, 0, t)),
            pl.BlockSpec((1, KNN, TILE), lambda t, b: (b, 0, t)),
            pl.BlockSpec((KNN, HID, TILE), lambda t, b: (0, 0, t)),
            pl.BlockSpec((HID, TILE), lambda t, b: (0, t)),
            pl.BlockSpec((TILE, PRJ), lambda t, b: (t, 0)),
            pl.BlockSpec((1, 6, 1024), lambda t, b: (b, 0, 0)),
            pl.BlockSpec((1, TILE, 1), lambda t, b: (b, t, 0)),
            _full((HID, 2)),
            _full((1024, PRJ)),
            _full((1, PRJ)),
            _full((6, 32)),
            _full((32, PRJ)),
            _full((1, PRJ)),
            _full((PRJ, PRJ)),
            _full((1, PRJ)),
            _full((PRJ, PRJ)),
            _full((1, PRJ)),
            _full((PRJ, PRJ)),
            _full((1, PRJ)),
            _full((1, HID + PRJ)),
            _full((1, HID + PRJ)),
            _full((HID + PRJ, HID)),
            _full((1, HID)),
            _full((HID, 2)),
            _full((1, 2)),
        ],
        out_specs=pl.BlockSpec((1, TILE, 2), lambda t, b: (b, t, 0)),
        out_shape=jax.ShapeDtypeStruct((B, N_PAD, 2), jnp.float32),
    )(gx0, gx1, gm, q2g, qlt, qg, latent_seq, mcol,
      W_nbr[:2].T, W_lat, b_lat.reshape(1, PRJ), face_emb, W_lf,
      b_lf.reshape(1, PRJ), W_k, b_k.reshape(1, PRJ), W_v,
      b_v.reshape(1, PRJ), W_go, b_go.reshape(1, PRJ),
      ln_g.reshape(1, HID + PRJ), ln_b.reshape(1, HID + PRJ),
      W_m1, b_m1.reshape(1, HID), W_m2, b_m2.reshape(1, 2))

    return out[:, :N_SENS, :]


# speedup vs baseline: 1.3021x; 1.3021x over previous
import functools

import jax
import jax.numpy as jnp
from jax import lax
from jax.experimental import pallas as pl
from jax.experimental.pallas import tpu as pltpu
from jax.experimental.pallas import tpu_sc as plsc

N_SENS = 4760
N_PAD = 5120
TILE = 512
NT = N_PAD // TILE
KNN = 16
HID = 64
PRJ = 128
NH = 4
HD = 32
NHALF = N_PAD // 2


def _prep(q_ref, wql_ref, bql_ref, wqg_ref, bqg_ref, ws_ref, bnbr_ref,
          qlt_out, qg_out, q2t_out):
    q = q_ref[...]
    qlt_out[...] = (q @ wql_ref[...] + bql_ref[...]).T
    qg_out[...] = q @ wqg_ref[...] + bqg_ref[...]
    q2t_out[...] = (q @ ws_ref[...] + bnbr_ref[...]).T


def _sc_gather(x0_hbm, x1_hbm, em_hbm, idx_hbm,
               gx0_hbm, gx1_hbm, gm_hbm,
               idx_v, s0_v, s1_v, s2_v, out_v):
    c = lax.axis_index("c")
    s = lax.axis_index("s")
    wid = s * 2 + c
    b = wid // 2
    n0 = (wid % 2) * NHALF
    pltpu.sync_copy(idx_hbm.at[:, pl.ds(n0, NHALF)], idx_v)
    pltpu.sync_copy(x0_hbm.at[b], s0_v)
    pltpu.sync_copy(x1_hbm.at[b], s1_v)
    pltpu.sync_copy(em_hbm.at[b], s2_v)

    def make_body(src_v, k):
        def body(j, carry):
            iv = idx_v[k, pl.ds(j * 16, 16)]
            out_v[k, pl.ds(j * 16, 16)] = plsc.load_gather(src_v, [iv])
            return carry
        return body

    for src_v, dst_hbm in ((s0_v, gx0_hbm), (s1_v, gx1_hbm), (s2_v, gm_hbm)):
        for k in range(KNN):
            lax.fori_loop(0, NHALF // 16, make_body(src_v, k), 0)
        pltpu.sync_copy(out_v, dst_hbm.at[b, :, pl.ds(n0, NHALF)])


def _sc_gatherq(q2t_hbm, idx_hbm, q2g_hbm, idx_v, s0_v, s1_v, out_v):
    c = lax.axis_index("c")
    s = lax.axis_index("s")
    h0 = (s * 2 + c) * 2
    pltpu.sync_copy(idx_hbm, idx_v)
    pltpu.sync_copy(q2t_hbm.at[h0], s0_v)
    pltpu.sync_copy(q2t_hbm.at[h0 + 1], s1_v)

    def make_body(k):
        def body(j, carry):
            iv = idx_v[k, pl.ds(j * 16, 16)]
            out_v[0, pl.ds(j * 16, 16)] = plsc.load_gather(s0_v, [iv])
            out_v[1, pl.ds(j * 16, 16)] = plsc.load_gather(s1_v, [iv])
            return carry
        return body

    for k in range(KNN):
        lax.fori_loop(0, N_PAD // 16, make_body(k), 0)
        pltpu.sync_copy(out_v, q2g_hbm.at[k, pl.ds(h0, 2), :])


def _main(gx0_ref, gx1_ref, gm_ref, q2g_ref, qlt_ref, qg_ref, lat_ref, msk_ref,
          wxt_ref, wlat_ref, blat_ref, femb_ref, wlf_ref, blf_ref,
          wk_ref, bk_ref, wv_ref, bv_ref, wgo_ref, bgo_ref,
          lng_ref, lnb_ref, wm1_ref, bm1_ref, wm2_ref, bm2_ref,
          out_ref):
    qlt = qlt_ref[...]
    w0c = wxt_ref[:, 0:1]
    w1c = wxt_ref[:, 1:2]
    a0 = jnp.sum(qlt * w0c, axis=0, keepdims=True)
    a1 = jnp.sum(qlt * w1c, axis=0, keepdims=True)
    rows = []
    for k in range(KNN):
        rows.append(jnp.sum(q2g_ref[k] * qlt, axis=0, keepdims=True))
    dq = jnp.concatenate(rows, axis=0)
    gx0 = gx0_ref[0]
    gx1 = gx1_ref[0]
    lg = (gx0 * a0 + gx1 * a1 + dq) * (HID ** -0.5)
    lg = jnp.where(gm_ref[0] > 0, -10000.0, lg)
    mx = jnp.max(lg, axis=0, keepdims=True)
    ex = jnp.exp(lg - mx)
    aw = ex / jnp.sum(ex, axis=0, keepdims=True)
    s0 = jnp.sum(aw * gx0, axis=0, keepdims=True)
    s1 = jnp.sum(aw * gx1, axis=0, keepdims=True)
    localt = w0c * s0 + w1c * s1
    for k in range(KNN):
        localt = localt + aw[k:k + 1, :] * q2g_ref[k]
    local = localt.T

    lat = lat_ref[0]
    lfb = femb_ref[...] @ wlf_ref[...] + blf_ref[...]
    kv = lat @ wlat_ref[...] + blat_ref[...] + lfb
    kg = kv @ wk_ref[...] + bk_ref[...]
    vg = kv @ wv_ref[...] + bv_ref[...]
    qg = qg_ref[...]
    heads = []
    for h in range(NH):
        qh = qg[:, HD * h:HD * (h + 1)]
        kh = kg[:, HD * h:HD * (h + 1)]
        vh = vg[:, HD * h:HD * (h + 1)]
        s = lax.dot_general(qh, kh, (((1,), (1,)), ((), ()))) * (HD ** -0.5)
        s = s - jnp.max(s, axis=1, keepdims=True)
        es = jnp.exp(s)
        hw = es / jnp.sum(es, axis=1, keepdims=True)
        heads.append(hw @ vh)
    gf = jnp.concatenate(heads, axis=1)
    gf = gf @ wgo_ref[...] + bgo_ref[...]

    comb = jnp.concatenate([local, gf], axis=1)
    mu = jnp.mean(comb, axis=1, keepdims=True)
    var = jnp.mean((comb - mu) * (comb - mu), axis=1, keepdims=True)
    xn = (comb - mu) * jax.lax.rsqrt(var + 1e-5) * lng_ref[...] + lnb_ref[...]
    hm = xn @ wm1_ref[...] + bm1_ref[...]
    hm = 0.5 * hm * (1.0 + jax.lax.erf(hm * (2.0 ** -0.5)))
    pr = hm @ wm2_ref[...] + bm2_ref[...]
    out_ref[0] = pr * msk_ref[0]


def _full(shape):
    nd = len(shape)
    return pl.BlockSpec(shape, lambda t, b, _n=nd: (0,) * _n)


def kernel(x_flat, latent_seq, mask, encoder_mask, pos_embed, knn_indices,
           face_ids, token_face_ids, face_emb, W_nbr, b_nbr, W_ql, b_ql,
           W_lat, b_lat, W_lf, b_lf, W_qg, b_qg, W_k, b_k, W_v, b_v,
           W_go, b_go, ln_g, ln_b, W_m1, b_m1, W_m2, b_m2):
    B = x_flat.shape[0]
    pad = N_PAD - N_SENS

    query = jnp.concatenate([pos_embed, face_emb[face_ids]], axis=-1)
    query = jnp.pad(query, ((0, pad), (0, 0)))

    qlt, qg, q2t = pl.pallas_call(
        _prep,
        grid=(NT,),
        in_specs=[
            pl.BlockSpec((TILE, 128), lambda t: (t, 0)),
            pl.BlockSpec((128, HID), lambda t: (0, 0)),
            pl.BlockSpec((1, HID), lambda t: (0, 0)),
            pl.BlockSpec((128, PRJ), lambda t: (0, 0)),
            pl.BlockSpec((1, PRJ), lambda t: (0, 0)),
            pl.BlockSpec((128, HID), lambda t: (0, 0)),
            pl.BlockSpec((1, HID), lambda t: (0, 0)),
        ],
        out_specs=[
            pl.BlockSpec((HID, TILE), lambda t: (0, t)),
            pl.BlockSpec((TILE, PRJ), lambda t: (t, 0)),
            pl.BlockSpec((HID, TILE), lambda t: (0, t)),
        ],
        out_shape=[
            jax.ShapeDtypeStruct((HID, N_PAD), jnp.float32),
            jax.ShapeDtypeStruct((N_PAD, PRJ), jnp.float32),
            jax.ShapeDtypeStruct((HID, N_PAD), jnp.float32),
        ],
    )(query, W_ql, b_ql.reshape(1, HID), W_qg, b_qg.reshape(1, PRJ),
      W_nbr[2:], b_nbr.reshape(1, HID))

    idxt = jnp.pad(knn_indices, ((0, pad), (0, 0))).T
    x0 = jnp.pad(x_flat[..., 0], ((0, 0), (0, pad)))
    x1 = jnp.pad(x_flat[..., 1], ((0, 0), (0, pad)))
    em = jnp.pad(encoder_mask, ((0, 0), (0, pad)))

    sc_gather = functools.partial(
        pl.kernel,
        out_type=[
            jax.ShapeDtypeStruct((B, KNN, N_PAD), jnp.float32),
            jax.ShapeDtypeStruct((B, KNN, N_PAD), jnp.float32),
            jax.ShapeDtypeStruct((B, KNN, N_PAD), jnp.float32),
        ],
        mesh=plsc.VectorSubcoreMesh(core_axis_name="c", subcore_axis_name="s"),
        compiler_params=pltpu.CompilerParams(needs_layout_passes=False),
        scratch_types=[
            pltpu.VMEM((KNN, NHALF), jnp.int32),
            pltpu.VMEM((N_PAD,), jnp.float32),
            pltpu.VMEM((N_PAD,), jnp.float32),
            pltpu.VMEM((N_PAD,), jnp.float32),
            pltpu.VMEM((KNN, NHALF), jnp.float32),
        ],
    )(_sc_gather)
    gx0, gx1, gm = sc_gather(x0, x1, em, idxt)

    sc_gatherq = functools.partial(
        pl.kernel,
        out_type=jax.ShapeDtypeStruct((KNN, HID, N_PAD), jnp.float32),
        mesh=plsc.VectorSubcoreMesh(core_axis_name="c", subcore_axis_name="s"),
        compiler_params=pltpu.CompilerParams(needs_layout_passes=False),
        scratch_types=[
            pltpu.VMEM((KNN, N_PAD), jnp.int32),
            pltpu.VMEM((N_PAD,), jnp.float32),
            pltpu.VMEM((N_PAD,), jnp.float32),
            pltpu.VMEM((2, N_PAD), jnp.float32),
        ],
    )(_sc_gatherq)
    q2g = sc_gatherq(q2t, idxt)

    mcol = jnp.pad(mask, ((0, 0), (0, pad)))[..., None]
    mcol = (mcol > 0).astype(jnp.float32)

    out = pl.pallas_call(
        _main,
        grid=(NT, B),
        in_specs=[
            pl.BlockSpec((1, KNN, TILE), lambda t, b: (b, 0, t)),
            pl.BlockSpec((1, KNN, TILE), lambda t, b: (b, 0, t)),
            pl.BlockSpec((1, KNN, TILE), lambda t, b: (b, 0, t)),
            pl.BlockSpec((KNN, HID, TILE), lambda t, b: (0, 0, t)),
            pl.BlockSpec((HID, TILE), lambda t, b: (0, t)),
            pl.BlockSpec((TILE, PRJ), lambda t, b: (t, 0)),
            pl.BlockSpec((1, 6, 1024), lambda t, b: (b, 0, 0)),
            pl.BlockSpec((1, TILE, 1), lambda t, b: (b, t, 0)),
            _full((HID, 2)),
            _full((1024, PRJ)),
            _full((1, PRJ)),
            _full((6, 32)),
            _full((32, PRJ)),
            _full((1, PRJ)),
            _full((PRJ, PRJ)),
            _full((1, PRJ)),
            _full((PRJ, PRJ)),
            _full((1, PRJ)),
            _full((PRJ, PRJ)),
            _full((1, PRJ)),
            _full((1, HID + PRJ)),
            _full((1, HID + PRJ)),
            _full((HID + PRJ, HID)),
            _full((1, HID)),
            _full((HID, 2)),
            _full((1, 2)),
        ],
        out_specs=pl.BlockSpec((1, TILE, 2), lambda t, b: (b, t, 0)),
        out_shape=jax.ShapeDtypeStruct((B, N_PAD, 2), jnp.float32),
    )(gx0, gx1, gm, q2g, qlt, qg, latent_seq, mcol,
      W_nbr[:2].T, W_lat, b_lat.reshape(1, PRJ), face_emb, W_lf,
      b_lf.reshape(1, PRJ), W_k, b_k.reshape(1, PRJ), W_v,
      b_v.reshape(1, PRJ), W_go, b_go.reshape(1, PRJ),
      ln_g.reshape(1, HID + PRJ), ln_b.reshape(1, HID + PRJ),
      W_m1, b_m1.reshape(1, HID), W_m2, b_m2.reshape(1, 2))

    return out[:, :N_SENS, :]


# speedup vs baseline: 1.5400x; 1.1827x over previous
import functools

import jax
import jax.numpy as jnp
from jax import lax
from jax.experimental import pallas as pl
from jax.experimental.pallas import tpu as pltpu
from jax.experimental.pallas import tpu_sc as plsc

N_SENS = 4760
N_PAD = 5120
TILE = 512
NT = N_PAD // TILE
KNN = 16
HID = 64
PRJ = 128
NH = 4
HD = 32
NHALF = N_PAD // 2


def _prep(q_ref, wql_ref, bql_ref, wqg_ref, bqg_ref, ws_ref, bnbr_ref,
          qlt_out, qg_out, q2t_out):
    q = q_ref[...]
    qlt_out[...] = (q @ wql_ref[...] + bql_ref[...]).T
    qg_out[...] = q @ wqg_ref[...] + bqg_ref[...]
    q2t_out[...] = (q @ ws_ref[...] + bnbr_ref[...]).T


def _sc_gather(x0_hbm, x1_hbm, em_hbm, idx_hbm,
               gx0_hbm, gx1_hbm, gm_hbm,
               idx_v, s0_v, s1_v, s2_v, out_v):
    c = lax.axis_index("c")
    s = lax.axis_index("s")
    wid = s * 2 + c
    b = wid // 2
    n0 = (wid % 2) * NHALF
    pltpu.sync_copy(idx_hbm.at[:, pl.ds(n0, NHALF)], idx_v)
    pltpu.sync_copy(x0_hbm.at[b], s0_v)
    pltpu.sync_copy(x1_hbm.at[b], s1_v)
    pltpu.sync_copy(em_hbm.at[b], s2_v)

    def make_body(src_v, k):
        def body(j, carry):
            for u in range(4):
                o = j * 64 + u * 16
                iv = idx_v[k, pl.ds(o, 16)]
                out_v[k, pl.ds(o, 16)] = plsc.load_gather(src_v, [iv])
            return carry
        return body

    for src_v, dst_hbm in ((s0_v, gx0_hbm), (s1_v, gx1_hbm), (s2_v, gm_hbm)):
        for k in range(KNN):
            lax.fori_loop(0, NHALF // 64, make_body(src_v, k), 0)
        pltpu.sync_copy(out_v, dst_hbm.at[b, :, pl.ds(n0, NHALF)])


def _sc_gatherq(q2t_hbm, idx_hbm, q2g_hbm, idx_v, s0_v, s1_v, out_v):
    c = lax.axis_index("c")
    s = lax.axis_index("s")
    h0 = (s * 2 + c) * 2
    pltpu.sync_copy(idx_hbm, idx_v)
    pltpu.sync_copy(q2t_hbm.at[h0], s0_v)
    pltpu.sync_copy(q2t_hbm.at[h0 + 1], s1_v)

    def make_body(k):
        def body(j, carry):
            for u in range(4):
                o = j * 64 + u * 16
                iv = idx_v[k, pl.ds(o, 16)]
                out_v[0, pl.ds(o, 16)] = plsc.load_gather(s0_v, [iv])
                out_v[1, pl.ds(o, 16)] = plsc.load_gather(s1_v, [iv])
            return carry
        return body

    for k in range(KNN):
        lax.fori_loop(0, N_PAD // 64, make_body(k), 0)
        pltpu.sync_copy(out_v, q2g_hbm.at[k, pl.ds(h0, 2), :])


def _main(gx0_ref, gx1_ref, gm_ref, q2g_ref, dq_ref, a01_ref, qg_ref,
          kg_ref, vg_ref, msk_ref, wxt_ref, wgo_ref, bgo_ref,
          lng_ref, lnb_ref, wm1_ref, bm1_ref, wm2_ref, bm2_ref,
          out_ref):
    gx0 = gx0_ref[0]
    gx1 = gx1_ref[0]
    lg = gx0 * a01_ref[0:1, :] + gx1 * a01_ref[1:2, :] + dq_ref[...]
    lg = jnp.where(gm_ref[0] > 0, -10000.0, lg)
    mx = jnp.max(lg, axis=0, keepdims=True)
    ex = jnp.exp(lg - mx)
    aw = ex / jnp.sum(ex, axis=0, keepdims=True)
    s0 = jnp.sum(aw * gx0, axis=0, keepdims=True)
    s1 = jnp.sum(aw * gx1, axis=0, keepdims=True)
    w0c = wxt_ref[:, 0:1]
    w1c = wxt_ref[:, 1:2]
    localt = w0c * s0 + w1c * s1
    for k in range(KNN):
        localt = localt + aw[k:k + 1, :] * q2g_ref[k]
    local = localt.T

    kg = kg_ref[0]
    vg = vg_ref[0]
    qg = qg_ref[...]
    ones6 = jnp.full((6, 1), 1.0, jnp.float32)
    heads = []
    for h in range(NH):
        qh = qg[:, HD * h:HD * (h + 1)]
        kh = kg[:, HD * h:HD * (h + 1)]
        vh = vg[:, HD * h:HD * (h + 1)]
        es = jnp.exp(lax.dot_general(qh, kh, (((1,), (1,)), ((), ()))))
        num = es @ vh
        den = es @ ones6
        heads.append(num / den)
    gf = jnp.concatenate(heads, axis=1)
    gf = gf @ wgo_ref[...] + bgo_ref[...]

    comb = jnp.concatenate([local, gf], axis=1)
    wmean = jnp.full((HID + PRJ, 1), 1.0 / (HID + PRJ), jnp.float32)
    mu = comb @ wmean
    d = comb - mu
    var = (d * d) @ wmean
    xn = d * jax.lax.rsqrt(var + 1e-5) * lng_ref[...] + lnb_ref[...]
    hm = xn @ wm1_ref[...] + bm1_ref[...]
    hm = 0.5 * hm * (1.0 + jax.lax.erf(hm * (2.0 ** -0.5)))
    pr = hm @ wm2_ref[...] + bm2_ref[...]
    out_ref[0] = pr * msk_ref[0]


def _dqprep(q2g_ref, qlt_ref, wxt_ref, dq_out, a01_out):
    qlt = qlt_ref[...]
    sc = HID ** -0.5
    rows = []
    for k in range(KNN):
        rows.append(jnp.sum(q2g_ref[k] * qlt, axis=0, keepdims=True))
    dq_out[...] = jnp.concatenate(rows, axis=0) * sc
    a01_out[0:1, :] = jnp.sum(qlt * wxt_ref[:, 0:1], axis=0, keepdims=True) * sc
    a01_out[1:2, :] = jnp.sum(qlt * wxt_ref[:, 1:2], axis=0, keepdims=True) * sc


def _kvprep(lat_ref, wlat_ref, blat_ref, femb_ref, wlf_ref, blf_ref,
            wk_ref, bk_ref, wv_ref, bv_ref, kg_out, vg_out):
    lat = lat_ref[0]
    lfb = femb_ref[...] @ wlf_ref[...] + blf_ref[...]
    kv = lat @ wlat_ref[...] + blat_ref[...] + lfb
    kg_out[0] = (kv @ wk_ref[...] + bk_ref[...]) * (HD ** -0.5)
    vg_out[0] = kv @ wv_ref[...] + bv_ref[...]


def _full(shape):
    nd = len(shape)
    return pl.BlockSpec(shape, lambda t, b, _n=nd: (0,) * _n)


def kernel(x_flat, latent_seq, mask, encoder_mask, pos_embed, knn_indices,
           face_ids, token_face_ids, face_emb, W_nbr, b_nbr, W_ql, b_ql,
           W_lat, b_lat, W_lf, b_lf, W_qg, b_qg, W_k, b_k, W_v, b_v,
           W_go, b_go, ln_g, ln_b, W_m1, b_m1, W_m2, b_m2):
    B = x_flat.shape[0]
    pad = N_PAD - N_SENS

    query = jnp.concatenate([pos_embed, face_emb[face_ids]], axis=-1)
    query = jnp.pad(query, ((0, pad), (0, 0)))

    qlt, qg, q2t = pl.pallas_call(
        _prep,
        grid=(NT,),
        in_specs=[
            pl.BlockSpec((TILE, 128), lambda t: (t, 0)),
            pl.BlockSpec((128, HID), lambda t: (0, 0)),
            pl.BlockSpec((1, HID), lambda t: (0, 0)),
            pl.BlockSpec((128, PRJ), lambda t: (0, 0)),
            pl.BlockSpec((1, PRJ), lambda t: (0, 0)),
            pl.BlockSpec((128, HID), lambda t: (0, 0)),
            pl.BlockSpec((1, HID), lambda t: (0, 0)),
        ],
        out_specs=[
            pl.BlockSpec((HID, TILE), lambda t: (0, t)),
            pl.BlockSpec((TILE, PRJ), lambda t: (t, 0)),
            pl.BlockSpec((HID, TILE), lambda t: (0, t)),
        ],
        out_shape=[
            jax.ShapeDtypeStruct((HID, N_PAD), jnp.float32),
            jax.ShapeDtypeStruct((N_PAD, PRJ), jnp.float32),
            jax.ShapeDtypeStruct((HID, N_PAD), jnp.float32),
        ],
    )(query, W_ql, b_ql.reshape(1, HID), W_qg, b_qg.reshape(1, PRJ),
      W_nbr[2:], b_nbr.reshape(1, HID))

    idxt = jnp.pad(knn_indices, ((0, pad), (0, 0))).T
    x0 = jnp.pad(x_flat[..., 0], ((0, 0), (0, pad)))
    x1 = jnp.pad(x_flat[..., 1], ((0, 0), (0, pad)))
    em = jnp.pad(encoder_mask, ((0, 0), (0, pad)))

    sc_gather = functools.partial(
        pl.kernel,
        out_type=[
            jax.ShapeDtypeStruct((B, KNN, N_PAD), jnp.float32),
            jax.ShapeDtypeStruct((B, KNN, N_PAD), jnp.float32),
            jax.ShapeDtypeStruct((B, KNN, N_PAD), jnp.float32),
        ],
        mesh=plsc.VectorSubcoreMesh(core_axis_name="c", subcore_axis_name="s"),
        compiler_params=pltpu.CompilerParams(needs_layout_passes=False),
        scratch_types=[
            pltpu.VMEM((KNN, NHALF), jnp.int32),
            pltpu.VMEM((N_PAD,), jnp.float32),
            pltpu.VMEM((N_PAD,), jnp.float32),
            pltpu.VMEM((N_PAD,), jnp.float32),
            pltpu.VMEM((KNN, NHALF), jnp.float32),
        ],
    )(_sc_gather)
    gx0, gx1, gm = sc_gather(x0, x1, em, idxt)

    sc_gatherq = functools.partial(
        pl.kernel,
        out_type=jax.ShapeDtypeStruct((KNN, HID, N_PAD), jnp.float32),
        mesh=plsc.VectorSubcoreMesh(core_axis_name="c", subcore_axis_name="s"),
        compiler_params=pltpu.CompilerParams(needs_layout_passes=False),
        scratch_types=[
            pltpu.VMEM((KNN, N_PAD), jnp.int32),
            pltpu.VMEM((N_PAD,), jnp.float32),
            pltpu.VMEM((N_PAD,), jnp.float32),
            pltpu.VMEM((2, N_PAD), jnp.float32),
        ],
    )(_sc_gatherq)
    q2g = sc_gatherq(q2t, idxt)

    wxt = W_nbr[:2].T
    dq, a01 = pl.pallas_call(
        _dqprep,
        grid=(NT,),
        in_specs=[
            pl.BlockSpec((KNN, HID, TILE), lambda t: (0, 0, t)),
            pl.BlockSpec((HID, TILE), lambda t: (0, t)),
            pl.BlockSpec((HID, 2), lambda t: (0, 0)),
        ],
        out_specs=[
            pl.BlockSpec((KNN, TILE), lambda t: (0, t)),
            pl.BlockSpec((2, TILE), lambda t: (0, t)),
        ],
        out_shape=[
            jax.ShapeDtypeStruct((KNN, N_PAD), jnp.float32),
            jax.ShapeDtypeStruct((2, N_PAD), jnp.float32),
        ],
    )(q2g, qlt, wxt)

    kg, vg = pl.pallas_call(
        _kvprep,
        grid=(B,),
        in_specs=[
            pl.BlockSpec((1, 6, 1024), lambda b: (b, 0, 0)),
            pl.BlockSpec((1024, PRJ), lambda b: (0, 0)),
            pl.BlockSpec((1, PRJ), lambda b: (0, 0)),
            pl.BlockSpec((6, 32), lambda b: (0, 0)),
            pl.BlockSpec((32, PRJ), lambda b: (0, 0)),
            pl.BlockSpec((1, PRJ), lambda b: (0, 0)),
            pl.BlockSpec((PRJ, PRJ), lambda b: (0, 0)),
            pl.BlockSpec((1, PRJ), lambda b: (0, 0)),
            pl.BlockSpec((PRJ, PRJ), lambda b: (0, 0)),
            pl.BlockSpec((1, PRJ), lambda b: (0, 0)),
        ],
        out_specs=[
            pl.BlockSpec((1, 6, PRJ), lambda b: (b, 0, 0)),
            pl.BlockSpec((1, 6, PRJ), lambda b: (b, 0, 0)),
        ],
        out_shape=[
            jax.ShapeDtypeStruct((B, 6, PRJ), jnp.float32),
            jax.ShapeDtypeStruct((B, 6, PRJ), jnp.float32),
        ],
    )(latent_seq, W_lat, b_lat.reshape(1, PRJ), face_emb, W_lf,
      b_lf.reshape(1, PRJ), W_k, b_k.reshape(1, PRJ), W_v,
      b_v.reshape(1, PRJ))

    mcol = jnp.pad(mask, ((0, 0), (0, pad)))[..., None]
    mcol = (mcol > 0).astype(jnp.float32)

    out = pl.pallas_call(
        _main,
        grid=(NT, B),
        in_specs=[
            pl.BlockSpec((1, KNN, TILE), lambda t, b: (b, 0, t)),
            pl.BlockSpec((1, KNN, TILE), lambda t, b: (b, 0, t)),
            pl.BlockSpec((1, KNN, TILE), lambda t, b: (b, 0, t)),
            pl.BlockSpec((KNN, HID, TILE), lambda t, b: (0, 0, t)),
            pl.BlockSpec((KNN, TILE), lambda t, b: (0, t)),
            pl.BlockSpec((2, TILE), lambda t, b: (0, t)),
            pl.BlockSpec((TILE, PRJ), lambda t, b: (t, 0)),
            pl.BlockSpec((1, 6, PRJ), lambda t, b: (b, 0, 0)),
            pl.BlockSpec((1, 6, PRJ), lambda t, b: (b, 0, 0)),
            pl.BlockSpec((1, TILE, 1), lambda t, b: (b, t, 0)),
            _full((HID, 2)),
            _full((PRJ, PRJ)),
            _full((1, PRJ)),
            _full((1, HID + PRJ)),
            _full((1, HID + PRJ)),
            _full((HID + PRJ, HID)),
            _full((1, HID)),
            _full((HID, 2)),
            _full((1, 2)),
        ],
        out_specs=pl.BlockSpec((1, TILE, 2), lambda t, b: (b, t, 0)),
        out_shape=jax.ShapeDtypeStruct((B, N_PAD, 2), jnp.float32),
    )(gx0, gx1, gm, q2g, dq, a01, qg, kg, vg, mcol,
      wxt, W_go, b_go.reshape(1, PRJ),
      ln_g.reshape(1, HID + PRJ), ln_b.reshape(1, HID + PRJ),
      W_m1, b_m1.reshape(1, HID), W_m2, b_m2.reshape(1, 2))

    return out[:, :N_SENS, :]


# speedup vs baseline: 1.6221x; 1.0533x over previous
import functools

import jax
import jax.numpy as jnp
from jax import lax
from jax.experimental import pallas as pl
from jax.experimental.pallas import tpu as pltpu
from jax.experimental.pallas import tpu_sc as plsc

N_SENS = 4760
N_PAD = 5120
TILE = 512
NT = N_PAD // TILE
KNN = 16
HID = 64
PRJ = 128
NH = 4
HD = 32
NHALF = N_PAD // 2


def _prep(q_ref, wql_ref, bql_ref, wqg_ref, bqg_ref, ws_ref, bnbr_ref,
          qlt_out, qg_out, q2t_out):
    q = q_ref[...]
    qlt_out[...] = (q @ wql_ref[...] + bql_ref[...]).T
    qg_out[...] = q @ wqg_ref[...] + bqg_ref[...]
    q2t_out[...] = (q @ ws_ref[...] + bnbr_ref[...]).T


def _sc_gather(x0_hbm, x1_hbm, em_hbm, idx_hbm,
               gx0_hbm, gx1_hbm, gm_hbm,
               idx_v, s0_v, s1_v, s2_v, out_v):
    c = lax.axis_index("c")
    s = lax.axis_index("s")
    wid = s * 2 + c
    b = wid // 2
    n0 = (wid % 2) * NHALF
    pltpu.sync_copy(idx_hbm.at[:, pl.ds(n0, NHALF)], idx_v)
    pltpu.sync_copy(x0_hbm.at[b], s0_v)
    pltpu.sync_copy(x1_hbm.at[b], s1_v)
    pltpu.sync_copy(em_hbm.at[b], s2_v)

    def make_body(src_v, k):
        def body(j, carry):
            for u in range(8):
                o = j * 128 + u * 16
                iv = idx_v[k, pl.ds(o, 16)]
                out_v[k, pl.ds(o, 16)] = plsc.load_gather(src_v, [iv])
            return carry
        return body

    for src_v, dst_hbm in ((s0_v, gx0_hbm), (s1_v, gx1_hbm), (s2_v, gm_hbm)):
        for k in range(KNN):
            lax.fori_loop(0, NHALF // 128, make_body(src_v, k), 0)
        pltpu.sync_copy(out_v, dst_hbm.at[b, :, pl.ds(n0, NHALF)])


def _sc_gatherq(q2t_hbm, idx_hbm, q2g_hbm, idx_v, s0_v, s1_v, out_v):
    c = lax.axis_index("c")
    s = lax.axis_index("s")
    h0 = (s * 2 + c) * 2
    pltpu.sync_copy(idx_hbm, idx_v)
    pltpu.sync_copy(q2t_hbm.at[h0], s0_v)
    pltpu.sync_copy(q2t_hbm.at[h0 + 1], s1_v)

    def make_body(k):
        def body(j, carry):
            for u in range(8):
                o = j * 128 + u * 16
                iv = idx_v[k, pl.ds(o, 16)]
                out_v[0, pl.ds(o, 16)] = plsc.load_gather(s0_v, [iv])
                out_v[1, pl.ds(o, 16)] = plsc.load_gather(s1_v, [iv])
            return carry
        return body

    for k in range(KNN):
        lax.fori_loop(0, N_PAD // 128, make_body(k), 0)
        pltpu.sync_copy(out_v, q2g_hbm.at[k, pl.ds(h0, 2), :])


def _main(gx0_ref, gx1_ref, gm_ref, q2g_ref, dq_ref, a01_ref, qg_ref,
          kg_ref, vg_ref, msk_ref, wxt_ref, wgo_ref, bgo_ref,
          lng_ref, lnb_ref, wm1_ref, bm1_ref, wm2_ref, bm2_ref,
          out_ref):
    gx0 = gx0_ref[0]
    gx1 = gx1_ref[0]
    lg = gx0 * a01_ref[0:1, :] + gx1 * a01_ref[1:2, :] + dq_ref[...]
    lg = jnp.where(gm_ref[0] > 0, -10000.0, lg)
    mx = jnp.max(lg, axis=0, keepdims=True)
    ex = jnp.exp(lg - mx)
    aw = ex / jnp.sum(ex, axis=0, keepdims=True)
    s0 = jnp.sum(aw * gx0, axis=0, keepdims=True)
    s1 = jnp.sum(aw * gx1, axis=0, keepdims=True)
    w0c = wxt_ref[:, 0:1]
    w1c = wxt_ref[:, 1:2]
    localt = w0c * s0 + w1c * s1
    for k in range(KNN):
        localt = localt + aw[k:k + 1, :] * q2g_ref[k]
    local = localt.T

    kg = kg_ref[0]
    vg = vg_ref[0]
    qg = qg_ref[...]
    ones6 = jnp.full((6, 1), 1.0, jnp.float32)
    heads = []
    for h in range(NH):
        qh = qg[:, HD * h:HD * (h + 1)]
        kh = kg[:, HD * h:HD * (h + 1)]
        vh = vg[:, HD * h:HD * (h + 1)]
        es = jnp.exp(lax.dot_general(qh, kh, (((1,), (1,)), ((), ()))))
        num = es @ vh
        den = es @ ones6
        heads.append(num / den)
    gf = jnp.concatenate(heads, axis=1)
    gf = gf @ wgo_ref[...] + bgo_ref[...]

    comb = jnp.concatenate([local, gf], axis=1)
    wmean = jnp.full((HID + PRJ, 1), 1.0 / (HID + PRJ), jnp.float32)
    mu = comb @ wmean
    d = comb - mu
    var = (d * d) @ wmean
    xn = d * jax.lax.rsqrt(var + 1e-5) * lng_ref[...] + lnb_ref[...]
    hm = xn @ wm1_ref[...] + bm1_ref[...]
    hm = 0.5 * hm * (1.0 + jax.lax.erf(hm * (2.0 ** -0.5)))
    pr = hm @ wm2_ref[...] + bm2_ref[...]
    out_ref[0] = pr * msk_ref[0]


def _dqprep(q2g_ref, qlt_ref, wxt_ref, dq_out, a01_out):
    qlt = qlt_ref[...]
    sc = HID ** -0.5
    rows = []
    for k in range(KNN):
        rows.append(jnp.sum(q2g_ref[k] * qlt, axis=0, keepdims=True))
    dq_out[...] = jnp.concatenate(rows, axis=0) * sc
    a01_out[0:1, :] = jnp.sum(qlt * wxt_ref[:, 0:1], axis=0, keepdims=True) * sc
    a01_out[1:2, :] = jnp.sum(qlt * wxt_ref[:, 1:2], axis=0, keepdims=True) * sc


def _kvprep(lat_ref, wlat_ref, blat_ref, femb_ref, wlf_ref, blf_ref,
            wk_ref, bk_ref, wv_ref, bv_ref, kg_out, vg_out):
    lat = lat_ref[0]
    lfb = femb_ref[...] @ wlf_ref[...] + blf_ref[...]
    kv = lat @ wlat_ref[...] + blat_ref[...] + lfb
    kg_out[0] = (kv @ wk_ref[...] + bk_ref[...]) * (HD ** -0.5)
    vg_out[0] = kv @ wv_ref[...] + bv_ref[...]


def _full(shape):
    nd = len(shape)
    return pl.BlockSpec(shape, lambda t, b, _n=nd: (0,) * _n)


def kernel(x_flat, latent_seq, mask, encoder_mask, pos_embed, knn_indices,
           face_ids, token_face_ids, face_emb, W_nbr, b_nbr, W_ql, b_ql,
           W_lat, b_lat, W_lf, b_lf, W_qg, b_qg, W_k, b_k, W_v, b_v,
           W_go, b_go, ln_g, ln_b, W_m1, b_m1, W_m2, b_m2):
    B = x_flat.shape[0]
    pad = N_PAD - N_SENS

    query = jnp.concatenate([pos_embed, face_emb[face_ids]], axis=-1)
    query = jnp.pad(query, ((0, pad), (0, 0)))

    qlt, qg, q2t = pl.pallas_call(
        _prep,
        grid=(NT,),
        in_specs=[
            pl.BlockSpec((TILE, 128), lambda t: (t, 0)),
            pl.BlockSpec((128, HID), lambda t: (0, 0)),
            pl.BlockSpec((1, HID), lambda t: (0, 0)),
            pl.BlockSpec((128, PRJ), lambda t: (0, 0)),
            pl.BlockSpec((1, PRJ), lambda t: (0, 0)),
            pl.BlockSpec((128, HID), lambda t: (0, 0)),
            pl.BlockSpec((1, HID), lambda t: (0, 0)),
        ],
        out_specs=[
            pl.BlockSpec((HID, TILE), lambda t: (0, t)),
            pl.BlockSpec((TILE, PRJ), lambda t: (t, 0)),
            pl.BlockSpec((HID, TILE), lambda t: (0, t)),
        ],
        out_shape=[
            jax.ShapeDtypeStruct((HID, N_PAD), jnp.float32),
            jax.ShapeDtypeStruct((N_PAD, PRJ), jnp.float32),
            jax.ShapeDtypeStruct((HID, N_PAD), jnp.float32),
        ],
    )(query, W_ql, b_ql.reshape(1, HID), W_qg, b_qg.reshape(1, PRJ),
      W_nbr[2:], b_nbr.reshape(1, HID))

    idxt = jnp.pad(knn_indices, ((0, pad), (0, 0))).T
    x0 = jnp.pad(x_flat[..., 0], ((0, 0), (0, pad)))
    x1 = jnp.pad(x_flat[..., 1], ((0, 0), (0, pad)))
    em = jnp.pad(encoder_mask, ((0, 0), (0, pad)))

    sc_gather = functools.partial(
        pl.kernel,
        out_type=[
            jax.ShapeDtypeStruct((B, KNN, N_PAD), jnp.float32),
            jax.ShapeDtypeStruct((B, KNN, N_PAD), jnp.float32),
            jax.ShapeDtypeStruct((B, KNN, N_PAD), jnp.float32),
        ],
        mesh=plsc.VectorSubcoreMesh(core_axis_name="c", subcore_axis_name="s"),
        compiler_params=pltpu.CompilerParams(needs_layout_passes=False),
        scratch_types=[
            pltpu.VMEM((KNN, NHALF), jnp.int32),
            pltpu.VMEM((N_PAD,), jnp.float32),
            pltpu.VMEM((N_PAD,), jnp.float32),
            pltpu.VMEM((N_PAD,), jnp.float32),
            pltpu.VMEM((KNN, NHALF), jnp.float32),
        ],
    )(_sc_gather)
    gx0, gx1, gm = sc_gather(x0, x1, em, idxt)

    sc_gatherq = functools.partial(
        pl.kernel,
        out_type=jax.ShapeDtypeStruct((KNN, HID, N_PAD), jnp.float32),
        mesh=plsc.VectorSubcoreMesh(core_axis_name="c", subcore_axis_name="s"),
        compiler_params=pltpu.CompilerParams(needs_layout_passes=False),
        scratch_types=[
            pltpu.VMEM((KNN, N_PAD), jnp.int32),
            pltpu.VMEM((N_PAD,), jnp.float32),
            pltpu.VMEM((N_PAD,), jnp.float32),
            pltpu.VMEM((2, N_PAD), jnp.float32),
        ],
    )(_sc_gatherq)
    q2g = sc_gatherq(q2t, idxt)

    wxt = W_nbr[:2].T
    dq, a01 = pl.pallas_call(
        _dqprep,
        grid=(NT,),
        in_specs=[
            pl.BlockSpec((KNN, HID, TILE), lambda t: (0, 0, t)),
            pl.BlockSpec((HID, TILE), lambda t: (0, t)),
            pl.BlockSpec((HID, 2), lambda t: (0, 0)),
        ],
        out_specs=[
            pl.BlockSpec((KNN, TILE), lambda t: (0, t)),
            pl.BlockSpec((2, TILE), lambda t: (0, t)),
        ],
        out_shape=[
            jax.ShapeDtypeStruct((KNN, N_PAD), jnp.float32),
            jax.ShapeDtypeStruct((2, N_PAD), jnp.float32),
        ],
    )(q2g, qlt, wxt)

    kg, vg = pl.pallas_call(
        _kvprep,
        grid=(B,),
        in_specs=[
            pl.BlockSpec((1, 6, 1024), lambda b: (b, 0, 0)),
            pl.BlockSpec((1024, PRJ), lambda b: (0, 0)),
            pl.BlockSpec((1, PRJ), lambda b: (0, 0)),
            pl.BlockSpec((6, 32), lambda b: (0, 0)),
            pl.BlockSpec((32, PRJ), lambda b: (0, 0)),
            pl.BlockSpec((1, PRJ), lambda b: (0, 0)),
            pl.BlockSpec((PRJ, PRJ), lambda b: (0, 0)),
            pl.BlockSpec((1, PRJ), lambda b: (0, 0)),
            pl.BlockSpec((PRJ, PRJ), lambda b: (0, 0)),
            pl.BlockSpec((1, PRJ), lambda b: (0, 0)),
        ],
        out_specs=[
            pl.BlockSpec((1, 6, PRJ), lambda b: (b, 0, 0)),
            pl.BlockSpec((1, 6, PRJ), lambda b: (b, 0, 0)),
        ],
        out_shape=[
            jax.ShapeDtypeStruct((B, 6, PRJ), jnp.float32),
            jax.ShapeDtypeStruct((B, 6, PRJ), jnp.float32),
        ],
    )(latent_seq, W_lat, b_lat.reshape(1, PRJ), face_emb, W_lf,
      b_lf.reshape(1, PRJ), W_k, b_k.reshape(1, PRJ), W_v,
      b_v.reshape(1, PRJ))

    mcol = jnp.pad(mask, ((0, 0), (0, pad)))[..., None]
    mcol = (mcol > 0).astype(jnp.float32)

    out = pl.pallas_call(
        _main,
        grid=(NT, B),
        in_specs=[
            pl.BlockSpec((1, KNN, TILE), lambda t, b: (b, 0, t)),
            pl.BlockSpec((1, KNN, TILE), lambda t, b: (b, 0, t)),
            pl.BlockSpec((1, KNN, TILE), lambda t, b: (b, 0, t)),
            pl.BlockSpec((KNN, HID, TILE), lambda t, b: (0, 0, t)),
            pl.BlockSpec((KNN, TILE), lambda t, b: (0, t)),
            pl.BlockSpec((2, TILE), lambda t, b: (0, t)),
            pl.BlockSpec((TILE, PRJ), lambda t, b: (t, 0)),
            pl.BlockSpec((1, 6, PRJ), lambda t, b: (b, 0, 0)),
            pl.BlockSpec((1, 6, PRJ), lambda t, b: (b, 0, 0)),
            pl.BlockSpec((1, TILE, 1), lambda t, b: (b, t, 0)),
            _full((HID, 2)),
            _full((PRJ, PRJ)),
            _full((1, PRJ)),
            _full((1, HID + PRJ)),
            _full((1, HID + PRJ)),
            _full((HID + PRJ, HID)),
            _full((1, HID)),
            _full((HID, 2)),
            _full((1, 2)),
        ],
        out_specs=pl.BlockSpec((1, TILE, 2), lambda t, b: (b, t, 0)),
        out_shape=jax.ShapeDtypeStruct((B, N_PAD, 2), jnp.float32),
    )(gx0, gx1, gm, q2g, dq, a01, qg, kg, vg, mcol,
      wxt, W_go, b_go.reshape(1, PRJ),
      ln_g.reshape(1, HID + PRJ), ln_b.reshape(1, HID + PRJ),
      W_m1, b_m1.reshape(1, HID), W_m2, b_m2.reshape(1, 2))

    return out[:, :N_SENS, :]
